# baseline (speedup 1.0000x reference)
import functools

import jax
import jax.numpy as jnp
from jax import lax
from jax.experimental import pallas as pl
from jax.experimental.pallas import tpu as pltpu
from jax.experimental.pallas import tpu_sc as plsc

NC, NS = 2, 16
NW = NC * NS
CHUNK = 80

_SC_MESH = dict(core_axis_name="c", subcore_axis_name="s", num_cores=NC,
                num_subcores=NS)


def _node_pre_body(h_ref, g_ref, b_ref, ws_ref, wd_ref, be1_ref,
                   hn_ref, a_ref, bb_ref):
    h = h_ref[...]
    mu = jnp.mean(h, axis=1, keepdims=True)
    var = jnp.mean((h - mu) ** 2, axis=1, keepdims=True)
    hn = (h - mu) * lax.rsqrt(var + 1e-5) * g_ref[...] + b_ref[...]
    hn_ref[...] = hn
    a_ref[...] = jnp.dot(hn, ws_ref[...], preferred_element_type=jnp.float32)
    bb_ref[...] = (jnp.dot(hn, wd_ref[...], preferred_element_type=jnp.float32)
                   + be1_ref[...])


def _node_pre(h, ln_g, ln_b, w_src, w_dst, be1, tn):
    n, d = h.shape
    grid = n // tn
    blk = lambda i: (i, 0)
    full = lambda i: (0, 0)
    out = jax.ShapeDtypeStruct((n, d), jnp.float32)
    return pl.pallas_call(
        _node_pre_body,
        grid=(grid,),
        in_specs=[pl.BlockSpec((tn, d), blk),
                  pl.BlockSpec((1, d), full), pl.BlockSpec((1, d), full),
                  pl.BlockSpec((d, d), full), pl.BlockSpec((d, d), full),
                  pl.BlockSpec((1, d), full)],
        out_specs=[pl.BlockSpec((tn, d), blk)] * 3,
        out_shape=[out, out, out],
    )(h, ln_g.reshape(1, d), ln_b.reshape(1, d), w_src, w_dst,
      be1.reshape(1, d))


def _gather_body(a_hbm, b_hbm, src_hbm, dst_hbm, zn_hbm, ones_hbm,
                 oa_hbm, ob_hbm, cnt_hbm,
                 si_v, di_v, ra_v, rb_v, ones_v, stage_v, acc_s,
                 sem_a, sem_b):
    e = src_hbm.shape[0]
    n = zn_hbm.shape[0]
    epw = e // NW
    c = lax.axis_index("c")
    s = lax.axis_index("s")
    wid = s * NC + c
    slab = n - (NS - 1) * 624
    nst = slab // 40

    def init(j, _):
        r = pl.ds(s * 624 + j * 40, 40)
        pltpu.sync_copy(zn_hbm.at[r], stage_v)
        pltpu.sync_copy(stage_v, acc_s.at[r])
        return 0

    lax.fori_loop(0, nst, init, 0)
    pltpu.sync_copy(ones_hbm, ones_v)
    plsc.subcore_barrier()

    base0 = wid * epw

    def body(k, _):
        base = base0 + k * CHUNK
        pltpu.sync_copy(src_hbm.at[pl.ds(base, CHUNK)], si_v)
        pltpu.sync_copy(dst_hbm.at[pl.ds(base, CHUNK)], di_v)
        cpa = pltpu.async_copy(a_hbm.at[si_v], ra_v, sem_a)
        cpb = pltpu.async_copy(b_hbm.at[di_v], rb_v, sem_b)
        pltpu.sync_copy(ones_v, acc_s.at[di_v], add=True)
        cpa.wait()
        cpb.wait()
        pltpu.sync_copy(ra_v, oa_hbm.at[pl.ds(base, CHUNK)])
        pltpu.sync_copy(rb_v, ob_hbm.at[pl.ds(base, CHUNK)])
        return 0

    lax.fori_loop(0, epw // CHUNK, body, 0)
    plsc.subcore_barrier()

    def dump(j, _):
        r = pl.ds(s * 624 + j * 40, 40)
        pltpu.sync_copy(acc_s.at[r], stage_v)
        pltpu.sync_copy(stage_v, cnt_hbm.at[c, r])
        return 0

    lax.fori_loop(0, nst, dump, 0)


def _gather(a, b, src, dst):
    n, d = a.shape
    e = src.shape[0]
    out = jax.ShapeDtypeStruct((e, d), jnp.float32)
    zn = jnp.zeros((n, d), jnp.float32)
    ones = jnp.ones((CHUNK, d), jnp.float32)
    fn = pl.kernel(
        _gather_body,
        out_type=(out, out, jax.ShapeDtypeStruct((NC, n, d), jnp.float32)),
        mesh=plsc.VectorSubcoreMesh(**_SC_MESH),
        scratch_types=[
            pltpu.VMEM((CHUNK,), jnp.int32),
            pltpu.VMEM((CHUNK,), jnp.int32),
            pltpu.VMEM((CHUNK, d), jnp.float32),
            pltpu.VMEM((CHUNK, d), jnp.float32),
            pltpu.VMEM((CHUNK, d), jnp.float32),
            pltpu.VMEM((40, d), jnp.float32),
            pltpu.VMEM_SHARED((n, d), jnp.float32),
            pltpu.SemaphoreType.DMA,
            pltpu.SemaphoreType.DMA,
        ],
    )
    return fn(a, b, src, dst, zn, ones)


def _edge_body(asrc_ref, bdst_ref, gram_ref, rbf_ref, wg_ref, wr_ref,
               we2_ref, be2_ref, m2_ref):
    x = asrc_ref[...] + bdst_ref[...]
    x = x + jnp.dot(gram_ref[...], wg_ref[...],
                    preferred_element_type=jnp.float32)
    x = x + jnp.dot(rbf_ref[...], wr_ref[...],
                    preferred_element_type=jnp.float32)
    m1 = x * jax.nn.sigmoid(x)
    y = jnp.dot(m1, we2_ref[...], preferred_element_type=jnp.float32) \
        + be2_ref[...]
    m2_ref[...] = y * jax.nn.sigmoid(y)


def _edge_mlp(asrc, bdst, gram, rbf, w_g, w_r, we2, be2, te):
    e, d = asrc.shape
    g = gram.shape[1]
    k = rbf.shape[1]
    grid = e // te
    blk = lambda i: (i, 0)
    full = lambda i: (0, 0)
    return pl.pallas_call(
        _edge_body,
        grid=(grid,),
        in_specs=[pl.BlockSpec((te, d), blk), pl.BlockSpec((te, d), blk),
                  pl.BlockSpec((te, g), blk), pl.BlockSpec((te, k), blk),
                  pl.BlockSpec((g, d), full), pl.BlockSpec((k, d), full),
                  pl.BlockSpec((d, d), full), pl.BlockSpec((1, d), full)],
        out_specs=pl.BlockSpec((te, d), blk),
        out_shape=jax.ShapeDtypeStruct((e, d), jnp.float32),
    )(asrc, bdst, gram, rbf, w_g, w_r, we2, be2.reshape(1, d))


def _scatter_sums_body(m2_hbm, dst_hbm, zn_hbm,
                       sums_hbm,
                       m2_v, idx_v, stage_v, acc_s, sem):
    e = dst_hbm.shape[0]
    n = zn_hbm.shape[0]
    epw = e // NW
    c = lax.axis_index("c")
    s = lax.axis_index("s")
    wid = s * NC + c
    slab = n - (NS - 1) * 624
    nst = slab // 40

    def init(j, _):
        r = pl.ds(s * 624 + j * 40, 40)
        pltpu.sync_copy(zn_hbm.at[r], stage_v)
        pltpu.sync_copy(stage_v, acc_s.at[r])
        return 0

    lax.fori_loop(0, nst, init, 0)
    plsc.subcore_barrier()

    base0 = wid * epw

    def body(k, _):
        base = base0 + k * CHUNK
        pltpu.sync_copy(dst_hbm.at[pl.ds(base, CHUNK)], idx_v)
        pltpu.sync_copy(m2_hbm.at[pl.ds(base, CHUNK)], m2_v)
        pltpu.sync_copy(m2_v, acc_s.at[idx_v], add=True)
        return 0

    lax.fori_loop(0, epw // CHUNK, body, 0)
    plsc.subcore_barrier()

    def dump(j, _):
        r = pl.ds(s * 624 + j * 40, 40)
        pltpu.sync_copy(acc_s.at[r], stage_v)
        pltpu.sync_copy(stage_v, sums_hbm.at[c, r])
        return 0

    lax.fori_loop(0, nst, dump, 0)


def _scatter_sums(m2, dst, n):
    e, d = m2.shape
    zn = jnp.zeros((n, d), jnp.float32)
    scratch = [
        pltpu.VMEM((CHUNK, d), jnp.float32),
        pltpu.VMEM((CHUNK,), jnp.int32),
        pltpu.VMEM((40, d), jnp.float32),
        pltpu.VMEM_SHARED((n, d), jnp.float32),
        pltpu.SemaphoreType.DMA,
    ]
    return pl.kernel(
        _scatter_sums_body,
        out_type=jax.ShapeDtypeStruct((NC, n, d), jnp.float32),
        mesh=plsc.VectorSubcoreMesh(**_SC_MESH),
        scratch_types=scratch,
    )(m2, dst, zn)


def _node_upd_body(h_ref, hn_ref, sums_ref, cnt_ref, w1h_ref, w1m_ref,
                   bn1_ref, wn2_ref, bn2_ref, out_ref):
    cnt = cnt_ref[0, :, 0:1] + cnt_ref[1, :, 0:1]
    m = (sums_ref[0] + sums_ref[1]) / jnp.maximum(cnt, 1.0)
    t = (jnp.dot(hn_ref[...], w1h_ref[...], preferred_element_type=jnp.float32)
         + jnp.dot(m, w1m_ref[...], preferred_element_type=jnp.float32)
         + bn1_ref[...])
    t = t * jax.nn.sigmoid(t)
    y = jnp.dot(t, wn2_ref[...], preferred_element_type=jnp.float32) \
        + bn2_ref[...]
    out_ref[...] = h_ref[...] + y * jax.nn.sigmoid(y)


def _node_update(h, hn, sums, cnt, w1h, w1m, bn1, wn2, bn2, tn):
    n, d = h.shape
    grid = n // tn
    blk = lambda i: (i, 0)
    blk3 = lambda i: (0, i, 0)
    full = lambda i: (0, 0)
    return pl.pallas_call(
        _node_upd_body,
        grid=(grid,),
        in_specs=[pl.BlockSpec((tn, d), blk), pl.BlockSpec((tn, d), blk),
                  pl.BlockSpec((NC, tn, d), blk3),
                  pl.BlockSpec((NC, tn, d), blk3),
                  pl.BlockSpec((d, d), full), pl.BlockSpec((d, d), full),
                  pl.BlockSpec((1, d), full), pl.BlockSpec((d, d), full),
                  pl.BlockSpec((1, d), full)],
        out_specs=pl.BlockSpec((tn, d), blk),
        out_shape=jax.ShapeDtypeStruct((n, d), jnp.float32),
    )(h, hn, sums, cnt, w1h, w1m, bn1.reshape(1, d), wn2, bn2.reshape(1, d))


def kernel(h, rbf_edge, gram_edge, edge_index, ln_g, ln_b,
           We1, be1, We2, be2, Wn1, bn1, Wn2, bn2):
    n, d = h.shape
    src = edge_index[0]
    dst = edge_index[1]
    g = gram_edge.shape[1]

    w_src = We1[:d]
    w_dst = We1[d:2 * d]
    w_g = We1[2 * d:2 * d + g]
    w_r = We1[2 * d + g:]

    hn, a_proj, b_proj = _node_pre(h, ln_g, ln_b, w_src, w_dst, be1, tn=1000)
    asrc, bdst, cnt = _gather(a_proj, b_proj, src, dst)
    m2 = _edge_mlp(asrc, bdst, gram_edge, rbf_edge, w_g, w_r, We2, be2,
                   te=2000)
    sums = _scatter_sums(m2, dst, n)
    return _node_update(h, hn, sums, cnt, Wn1[:d], Wn1[d:], bn1, Wn2, bn2,
                        tn=1000)


# speedup vs baseline: 1.2529x; 1.2529x over previous
import functools

import jax
import jax.numpy as jnp
from jax import lax
from jax.experimental import pallas as pl
from jax.experimental.pallas import tpu as pltpu
from jax.experimental.pallas import tpu_sc as plsc

NC, NS = 2, 16
NW = NC * NS
CHUNK = 80

_SC_MESH = dict(core_axis_name="c", subcore_axis_name="s", num_cores=NC,
                num_subcores=NS)


def _node_pre_body(h_ref, g_ref, b_ref, ws_ref, wd_ref, be1_ref,
                   hn_ref, a_ref, bb_ref):
    h = h_ref[...]
    mu = jnp.mean(h, axis=1, keepdims=True)
    var = jnp.mean((h - mu) ** 2, axis=1, keepdims=True)
    hn = (h - mu) * lax.rsqrt(var + 1e-5) * g_ref[...] + b_ref[...]
    hn_ref[...] = hn
    a_ref[...] = jnp.dot(hn, ws_ref[...], preferred_element_type=jnp.float32)
    bb_ref[...] = (jnp.dot(hn, wd_ref[...], preferred_element_type=jnp.float32)
                   + be1_ref[...])


def _node_pre(h, ln_g, ln_b, w_src, w_dst, be1, tn):
    n, d = h.shape
    grid = n // tn
    blk = lambda i: (i, 0)
    full = lambda i: (0, 0)
    out = jax.ShapeDtypeStruct((n, d), jnp.float32)
    return pl.pallas_call(
        _node_pre_body,
        grid=(grid,),
        in_specs=[pl.BlockSpec((tn, d), blk),
                  pl.BlockSpec((1, d), full), pl.BlockSpec((1, d), full),
                  pl.BlockSpec((d, d), full), pl.BlockSpec((d, d), full),
                  pl.BlockSpec((1, d), full)],
        out_specs=[pl.BlockSpec((tn, d), blk)] * 3,
        out_shape=[out, out, out],
    )(h, ln_g.reshape(1, d), ln_b.reshape(1, d), w_src, w_dst,
      be1.reshape(1, d))


def _gather_body(a_hbm, b_hbm, src_hbm, dst_hbm, oa_hbm, ob_hbm,
                 si_v, di_v, ra0_v, rb0_v, ra1_v, rb1_v,
                 sa0, sb0, sa1, sb1, swa, swb):
    e = src_hbm.shape[0]
    epw = e // NW
    wid = lax.axis_index("s") * NC + lax.axis_index("c")
    base0 = wid * epw

    pltpu.sync_copy(src_hbm.at[pl.ds(base0, epw)], si_v)
    pltpu.sync_copy(dst_hbm.at[pl.ds(base0, epw)], di_v)

    nch = epw // CHUNK

    def pair(kk, _):
        k = kk * 2
        base = base0 + k * CHUNK
        cpa0 = pltpu.async_copy(
            a_hbm.at[si_v.at[pl.ds(k * CHUNK, CHUNK)]], ra0_v, sa0)
        cpb0 = pltpu.async_copy(
            b_hbm.at[di_v.at[pl.ds(k * CHUNK, CHUNK)]], rb0_v, sb0)
        cpa1 = pltpu.async_copy(
            a_hbm.at[si_v.at[pl.ds((k + 1) * CHUNK, CHUNK)]], ra1_v, sa1)
        cpb1 = pltpu.async_copy(
            b_hbm.at[di_v.at[pl.ds((k + 1) * CHUNK, CHUNK)]], rb1_v, sb1)
        cpa0.wait()
        wa0 = pltpu.async_copy(ra0_v, oa_hbm.at[pl.ds(base, CHUNK)], swa)
        cpb0.wait()
        wb0 = pltpu.async_copy(rb0_v, ob_hbm.at[pl.ds(base, CHUNK)], swb)
        cpa1.wait()
        cpb1.wait()
        wa0.wait()
        wb0.wait()
        wa1 = pltpu.async_copy(
            ra1_v, oa_hbm.at[pl.ds(base + CHUNK, CHUNK)], swa)
        wb1 = pltpu.async_copy(
            rb1_v, ob_hbm.at[pl.ds(base + CHUNK, CHUNK)], swb)
        wa1.wait()
        wb1.wait()
        return 0

    lax.fori_loop(0, nch // 2, pair, 0)

    k = nch - 1
    base = base0 + k * CHUNK
    cpa0 = pltpu.async_copy(
        a_hbm.at[si_v.at[pl.ds(k * CHUNK, CHUNK)]], ra0_v, sa0)
    cpb0 = pltpu.async_copy(
        b_hbm.at[di_v.at[pl.ds(k * CHUNK, CHUNK)]], rb0_v, sb0)
    cpa0.wait()
    cpb0.wait()
    wa0 = pltpu.async_copy(ra0_v, oa_hbm.at[pl.ds(base, CHUNK)], swa)
    wb0 = pltpu.async_copy(rb0_v, ob_hbm.at[pl.ds(base, CHUNK)], swb)
    wa0.wait()
    wb0.wait()


def _gather(a, b, src, dst):
    n, d = a.shape
    e = src.shape[0]
    epw = e // NW
    out = jax.ShapeDtypeStruct((e, d), jnp.float32)
    fn = pl.kernel(
        _gather_body,
        out_type=(out, out),
        mesh=plsc.VectorSubcoreMesh(**_SC_MESH),
        scratch_types=[
            pltpu.VMEM((epw,), jnp.int32),
            pltpu.VMEM((epw,), jnp.int32),
            pltpu.VMEM((CHUNK, d), jnp.float32),
            pltpu.VMEM((CHUNK, d), jnp.float32),
            pltpu.VMEM((CHUNK, d), jnp.float32),
            pltpu.VMEM((CHUNK, d), jnp.float32),
            pltpu.SemaphoreType.DMA,
            pltpu.SemaphoreType.DMA,
            pltpu.SemaphoreType.DMA,
            pltpu.SemaphoreType.DMA,
            pltpu.SemaphoreType.DMA,
            pltpu.SemaphoreType.DMA,
        ],
    )
    return fn(a, b, src, dst)


def _edge_body(asrc_ref, bdst_ref, gram_ref, rbf_ref, wg_ref, wr_ref,
               we2_ref, be2_ref, m2_ref):
    x = asrc_ref[...] + bdst_ref[...]
    x = x + jnp.dot(gram_ref[...], wg_ref[...],
                    preferred_element_type=jnp.float32)
    x = x + jnp.dot(rbf_ref[...], wr_ref[...],
                    preferred_element_type=jnp.float32)
    m1 = x * jax.nn.sigmoid(x)
    y = jnp.dot(m1, we2_ref[...], preferred_element_type=jnp.float32) \
        + be2_ref[...]
    m2_ref[...] = y * jax.nn.sigmoid(y)


def _edge_mlp(asrc, bdst, gram, rbf, w_g, w_r, we2, be2, te):
    e, d = asrc.shape
    g = gram.shape[1]
    k = rbf.shape[1]
    grid = e // te
    blk = lambda i: (i, 0)
    full = lambda i: (0, 0)
    return pl.pallas_call(
        _edge_body,
        grid=(grid,),
        in_specs=[pl.BlockSpec((te, d), blk), pl.BlockSpec((te, d), blk),
                  pl.BlockSpec((te, g), blk), pl.BlockSpec((te, k), blk),
                  pl.BlockSpec((g, d), full), pl.BlockSpec((k, d), full),
                  pl.BlockSpec((d, d), full), pl.BlockSpec((1, d), full)],
        out_specs=pl.BlockSpec((te, d), blk),
        out_shape=jax.ShapeDtypeStruct((e, d), jnp.float32),
    )(asrc, bdst, gram, rbf, w_g, w_r, we2, be2.reshape(1, d))


def _acc_init(zn_hbm, acc_s, stage_v, s, n):
    slab = n - (NS - 1) * 624
    nst = slab // 40

    def init(j, _):
        r = pl.ds(s * 624 + j * 40, 40)
        pltpu.sync_copy(zn_hbm.at[r], stage_v)
        pltpu.sync_copy(stage_v, acc_s.at[r])
        return 0

    lax.fori_loop(0, nst, init, 0)


def _acc_dump(acc_s, out_hbm, stage_v, c, s, n):
    slab = n - (NS - 1) * 624
    nst = slab // 40

    def dump(j, _):
        r = pl.ds(s * 624 + j * 40, 40)
        pltpu.sync_copy(acc_s.at[r], stage_v)
        pltpu.sync_copy(stage_v, out_hbm.at[c, r])
        return 0

    lax.fori_loop(0, nst, dump, 0)


def _scatter_sums_body(m2_hbm, dst_hbm, zn_hbm,
                       sums_hbm,
                       m20_v, m21_v, di0_v, di1_v, stage_v, acc_s,
                       sm0, sm1, si0, si1, sc0, sc1):
    e = dst_hbm.shape[0]
    n = zn_hbm.shape[0]
    epw = e // NW
    c = lax.axis_index("c")
    s = lax.axis_index("s")
    wid = s * NC + c
    _acc_init(zn_hbm, acc_s, stage_v, s, n)
    plsc.subcore_barrier()

    base0 = wid * epw
    nch = epw // CHUNK

    def pair(kk, _):
        base = base0 + kk * 2 * CHUNK
        ci0 = pltpu.async_copy(dst_hbm.at[pl.ds(base, CHUNK)], di0_v, si0)
        ci1 = pltpu.async_copy(
            dst_hbm.at[pl.ds(base + CHUNK, CHUNK)], di1_v, si1)
        cm0 = pltpu.async_copy(m2_hbm.at[pl.ds(base, CHUNK)], m20_v, sm0)
        cm1 = pltpu.async_copy(
            m2_hbm.at[pl.ds(base + CHUNK, CHUNK)], m21_v, sm1)
        ci0.wait()
        cm0.wait()
        a0 = pltpu.async_copy(m20_v, acc_s.at[di0_v], sc0, add=True)
        ci1.wait()
        cm1.wait()
        a1 = pltpu.async_copy(m21_v, acc_s.at[di1_v], sc1, add=True)
        a0.wait()
        a1.wait()
        return 0

    lax.fori_loop(0, nch // 2, pair, 0)

    base = base0 + (nch - 1) * CHUNK
    pltpu.sync_copy(dst_hbm.at[pl.ds(base, CHUNK)], di0_v)
    pltpu.sync_copy(m2_hbm.at[pl.ds(base, CHUNK)], m20_v)
    pltpu.sync_copy(m20_v, acc_s.at[di0_v], add=True)

    plsc.subcore_barrier()
    _acc_dump(acc_s, sums_hbm, stage_v, c, s, n)


def _scatter_counts_body(dst_hbm, zn_hbm, ones_hbm,
                         cnt_hbm,
                         ones_v, di0_v, di1_v, stage_v, acc_s,
                         si0, si1, sc0, sc1):
    e = dst_hbm.shape[0]
    n = zn_hbm.shape[0]
    epw = e // NW
    c = lax.axis_index("c")
    s = lax.axis_index("s")
    wid = s * NC + c
    _acc_init(zn_hbm, acc_s, stage_v, s, n)
    pltpu.sync_copy(ones_hbm, ones_v)
    plsc.subcore_barrier()

    base0 = wid * epw
    nch = epw // CHUNK

    def pair(kk, _):
        base = base0 + kk * 2 * CHUNK
        ci0 = pltpu.async_copy(dst_hbm.at[pl.ds(base, CHUNK)], di0_v, si0)
        ci1 = pltpu.async_copy(
            dst_hbm.at[pl.ds(base + CHUNK, CHUNK)], di1_v, si1)
        ci0.wait()
        a0 = pltpu.async_copy(ones_v, acc_s.at[di0_v], sc0, add=True)
        ci1.wait()
        a1 = pltpu.async_copy(ones_v, acc_s.at[di1_v], sc1, add=True)
        a0.wait()
        a1.wait()
        return 0

    lax.fori_loop(0, nch // 2, pair, 0)

    base = base0 + (nch - 1) * CHUNK
    pltpu.sync_copy(dst_hbm.at[pl.ds(base, CHUNK)], di0_v)
    pltpu.sync_copy(ones_v, acc_s.at[di0_v], add=True)

    plsc.subcore_barrier()
    _acc_dump(acc_s, cnt_hbm, stage_v, c, s, n)


def _scatter_sums(m2, dst, n):
    e, d = m2.shape
    zn = jnp.zeros((n, d), jnp.float32)
    return pl.kernel(
        _scatter_sums_body,
        out_type=jax.ShapeDtypeStruct((NC, n, d), jnp.float32),
        mesh=plsc.VectorSubcoreMesh(**_SC_MESH),
        scratch_types=[
            pltpu.VMEM((CHUNK, d), jnp.float32),
            pltpu.VMEM((CHUNK, d), jnp.float32),
            pltpu.VMEM((CHUNK,), jnp.int32),
            pltpu.VMEM((CHUNK,), jnp.int32),
            pltpu.VMEM((40, d), jnp.float32),
            pltpu.VMEM_SHARED((n, d), jnp.float32),
            pltpu.SemaphoreType.DMA,
            pltpu.SemaphoreType.DMA,
            pltpu.SemaphoreType.DMA,
            pltpu.SemaphoreType.DMA,
            pltpu.SemaphoreType.DMA,
            pltpu.SemaphoreType.DMA,
        ],
    )(m2, dst, zn)


def _scatter_counts(dst, n, d):
    zn = jnp.zeros((n, d), jnp.float32)
    ones = jnp.ones((CHUNK, d), jnp.float32)
    return pl.kernel(
        _scatter_counts_body,
        out_type=jax.ShapeDtypeStruct((NC, n, d), jnp.float32),
        mesh=plsc.VectorSubcoreMesh(**_SC_MESH),
        scratch_types=[
            pltpu.VMEM((CHUNK, d), jnp.float32),
            pltpu.VMEM((CHUNK,), jnp.int32),
            pltpu.VMEM((CHUNK,), jnp.int32),
            pltpu.VMEM((40, d), jnp.float32),
            pltpu.VMEM_SHARED((n, d), jnp.float32),
            pltpu.SemaphoreType.DMA,
            pltpu.SemaphoreType.DMA,
            pltpu.SemaphoreType.DMA,
            pltpu.SemaphoreType.DMA,
        ],
    )(dst, zn, ones)


def _node_upd_body(h_ref, hn_ref, sums_ref, cnt_ref, w1h_ref, w1m_ref,
                   bn1_ref, wn2_ref, bn2_ref, out_ref):
    cnt = cnt_ref[0, :, 0:1] + cnt_ref[1, :, 0:1]
    m = (sums_ref[0] + sums_ref[1]) / jnp.maximum(cnt, 1.0)
    t = (jnp.dot(hn_ref[...], w1h_ref[...], preferred_element_type=jnp.float32)
         + jnp.dot(m, w1m_ref[...], preferred_element_type=jnp.float32)
         + bn1_ref[...])
    t = t * jax.nn.sigmoid(t)
    y = jnp.dot(t, wn2_ref[...], preferred_element_type=jnp.float32) \
        + bn2_ref[...]
    out_ref[...] = h_ref[...] + y * jax.nn.sigmoid(y)


def _node_update(h, hn, sums, cnt, w1h, w1m, bn1, wn2, bn2, tn):
    n, d = h.shape
    grid = n // tn
    blk = lambda i: (i, 0)
    blk3 = lambda i: (0, i, 0)
    full = lambda i: (0, 0)
    return pl.pallas_call(
        _node_upd_body,
        grid=(grid,),
        in_specs=[pl.BlockSpec((tn, d), blk), pl.BlockSpec((tn, d), blk),
                  pl.BlockSpec((NC, tn, d), blk3),
                  pl.BlockSpec((NC, tn, d), blk3),
                  pl.BlockSpec((d, d), full), pl.BlockSpec((d, d), full),
                  pl.BlockSpec((1, d), full), pl.BlockSpec((d, d), full),
                  pl.BlockSpec((1, d), full)],
        out_specs=pl.BlockSpec((tn, d), blk),
        out_shape=jax.ShapeDtypeStruct((n, d), jnp.float32),
    )(h, hn, sums, cnt, w1h, w1m, bn1.reshape(1, d), wn2, bn2.reshape(1, d))


def kernel(h, rbf_edge, gram_edge, edge_index, ln_g, ln_b,
           We1, be1, We2, be2, Wn1, bn1, Wn2, bn2):
    n, d = h.shape
    src = edge_index[0]
    dst = edge_index[1]
    g = gram_edge.shape[1]

    w_src = We1[:d]
    w_dst = We1[d:2 * d]
    w_g = We1[2 * d:2 * d + g]
    w_r = We1[2 * d + g:]

    hn, a_proj, b_proj = _node_pre(h, ln_g, ln_b, w_src, w_dst, be1, tn=1000)
    asrc, bdst = _gather(a_proj, b_proj, src, dst)
    cnt = _scatter_counts(dst, n, d)
    m2 = _edge_mlp(asrc, bdst, gram_edge, rbf_edge, w_g, w_r, We2, be2,
                   te=2000)
    sums = _scatter_sums(m2, dst, n)
    return _node_update(h, hn, sums, cnt, Wn1[:d], Wn1[d:], bn1, Wn2, bn2,
                        tn=1000)


# speedup vs baseline: 1.3886x; 1.1084x over previous
import functools

import jax
import jax.numpy as jnp
from jax import lax
from jax.experimental import pallas as pl
from jax.experimental.pallas import tpu as pltpu
from jax.experimental.pallas import tpu_sc as plsc

NC, NS = 2, 16
NW = NC * NS
CHUNK = 80

_SC_MESH = dict(core_axis_name="c", subcore_axis_name="s", num_cores=NC,
                num_subcores=NS)


def _node_pre_body(h_ref, g_ref, b_ref, ws_ref, wd_ref, be1_ref,
                   hn_ref, a_ref, bb_ref):
    h = h_ref[...]
    mu = jnp.mean(h, axis=1, keepdims=True)
    var = jnp.mean((h - mu) ** 2, axis=1, keepdims=True)
    hn = (h - mu) * lax.rsqrt(var + 1e-5) * g_ref[...] + b_ref[...]
    hn_ref[...] = hn
    a_ref[...] = jnp.dot(hn, ws_ref[...], preferred_element_type=jnp.float32)
    bb_ref[...] = (jnp.dot(hn, wd_ref[...], preferred_element_type=jnp.float32)
                   + be1_ref[...])


def _node_pre(h, ln_g, ln_b, w_src, w_dst, be1, tn):
    n, d = h.shape
    grid = n // tn
    blk = lambda i: (i, 0)
    full = lambda i: (0, 0)
    out = jax.ShapeDtypeStruct((n, d), jnp.float32)
    return pl.pallas_call(
        _node_pre_body,
        grid=(grid,),
        in_specs=[pl.BlockSpec((tn, d), blk),
                  pl.BlockSpec((1, d), full), pl.BlockSpec((1, d), full),
                  pl.BlockSpec((d, d), full), pl.BlockSpec((d, d), full),
                  pl.BlockSpec((1, d), full)],
        out_specs=[pl.BlockSpec((tn, d), blk)] * 3,
        out_shape=[out, out, out],
    )(h, ln_g.reshape(1, d), ln_b.reshape(1, d), w_src, w_dst,
      be1.reshape(1, d))


def _row_add(dst_ref, src_ref, d):
    def row(i, _):
        for j in range(d // 16):
            sl = pl.ds(j * 16, 16)
            dst_ref[i, sl] = dst_ref[i, sl] + src_ref[i, sl]
        return 0

    lax.fori_loop(0, CHUNK, row, 0)


def _gather_body(a_hbm, b_hbm, src_hbm, dst_hbm, og_hbm,
                 si_v, di_v, ra0_v, rb0_v, ra1_v, rb1_v,
                 sa0, sb0, sa1, sb1, swa, swb):
    e = src_hbm.shape[0]
    d = a_hbm.shape[1]
    epw = e // NW
    wid = lax.axis_index("s") * NC + lax.axis_index("c")
    base0 = wid * epw

    pltpu.sync_copy(src_hbm.at[pl.ds(base0, epw)], si_v)
    pltpu.sync_copy(dst_hbm.at[pl.ds(base0, epw)], di_v)

    nch = epw // CHUNK

    def pair(kk, _):
        k = kk * 2
        base = base0 + k * CHUNK
        cpa0 = pltpu.async_copy(
            a_hbm.at[si_v.at[pl.ds(k * CHUNK, CHUNK)]], ra0_v, sa0)
        cpb0 = pltpu.async_copy(
            b_hbm.at[di_v.at[pl.ds(k * CHUNK, CHUNK)]], rb0_v, sb0)
        cpa1 = pltpu.async_copy(
            a_hbm.at[si_v.at[pl.ds((k + 1) * CHUNK, CHUNK)]], ra1_v, sa1)
        cpb1 = pltpu.async_copy(
            b_hbm.at[di_v.at[pl.ds((k + 1) * CHUNK, CHUNK)]], rb1_v, sb1)
        cpa0.wait()
        cpb0.wait()
        _row_add(ra0_v, rb0_v, d)
        wa0 = pltpu.async_copy(ra0_v, og_hbm.at[pl.ds(base, CHUNK)], swa)
        cpa1.wait()
        cpb1.wait()
        _row_add(ra1_v, rb1_v, d)
        wb1 = pltpu.async_copy(
            ra1_v, og_hbm.at[pl.ds(base + CHUNK, CHUNK)], swb)
        wa0.wait()
        wb1.wait()
        return 0

    lax.fori_loop(0, nch // 2, pair, 0)

    k = nch - 1
    base = base0 + k * CHUNK
    cpa0 = pltpu.async_copy(
        a_hbm.at[si_v.at[pl.ds(k * CHUNK, CHUNK)]], ra0_v, sa0)
    cpb0 = pltpu.async_copy(
        b_hbm.at[di_v.at[pl.ds(k * CHUNK, CHUNK)]], rb0_v, sb0)
    cpa0.wait()
    cpb0.wait()
    _row_add(ra0_v, rb0_v, d)
    wa0 = pltpu.async_copy(ra0_v, og_hbm.at[pl.ds(base, CHUNK)], swa)
    wa0.wait()


def _gather(a, b, src, dst):
    n, d = a.shape
    e = src.shape[0]
    epw = e // NW
    fn = pl.kernel(
        _gather_body,
        out_type=jax.ShapeDtypeStruct((e, d), jnp.float32),
        mesh=plsc.VectorSubcoreMesh(**_SC_MESH),
        scratch_types=[
            pltpu.VMEM((epw,), jnp.int32),
            pltpu.VMEM((epw,), jnp.int32),
            pltpu.VMEM((CHUNK, d), jnp.float32),
            pltpu.VMEM((CHUNK, d), jnp.float32),
            pltpu.VMEM((CHUNK, d), jnp.float32),
            pltpu.VMEM((CHUNK, d), jnp.float32),
            pltpu.SemaphoreType.DMA,
            pltpu.SemaphoreType.DMA,
            pltpu.SemaphoreType.DMA,
            pltpu.SemaphoreType.DMA,
            pltpu.SemaphoreType.DMA,
            pltpu.SemaphoreType.DMA,
        ],
    )
    return fn(a, b, src, dst)


def _edge_body(g_ref, gram_ref, rbf_ref, wg_ref, wr_ref,
               we2_ref, be2_ref, m2_ref):
    x = g_ref[...]
    x = x + jnp.dot(gram_ref[...], wg_ref[...],
                    preferred_element_type=jnp.float32)
    x = x + jnp.dot(rbf_ref[...], wr_ref[...],
                    preferred_element_type=jnp.float32)
    m1 = x * jax.nn.sigmoid(x)
    y = jnp.dot(m1, we2_ref[...], preferred_element_type=jnp.float32) \
        + be2_ref[...]
    m2_ref[...] = y * jax.nn.sigmoid(y)


def _edge_mlp(gsum, gram, rbf, w_g, w_r, we2, be2, te):
    e, d = gsum.shape
    g = gram.shape[1]
    k = rbf.shape[1]
    grid = e // te
    blk = lambda i: (i, 0)
    full = lambda i: (0, 0)
    return pl.pallas_call(
        _edge_body,
        grid=(grid,),
        in_specs=[pl.BlockSpec((te, d), blk),
                  pl.BlockSpec((te, g), blk), pl.BlockSpec((te, k), blk),
                  pl.BlockSpec((g, d), full), pl.BlockSpec((k, d), full),
                  pl.BlockSpec((d, d), full), pl.BlockSpec((1, d), full)],
        out_specs=pl.BlockSpec((te, d), blk),
        out_shape=jax.ShapeDtypeStruct((e, d), jnp.float32),
    )(gsum, gram, rbf, w_g, w_r, we2, be2.reshape(1, d))


def _acc_init(zn_hbm, acc_s, stage_v, s, n):
    slab = n - (NS - 1) * 624
    nst = slab // 40

    def init(j, _):
        r = pl.ds(s * 624 + j * 40, 40)
        pltpu.sync_copy(zn_hbm.at[r], stage_v)
        pltpu.sync_copy(stage_v, acc_s.at[r])
        return 0

    lax.fori_loop(0, nst, init, 0)


def _acc_dump(acc_s, out_hbm, stage_v, c, s, n):
    slab = n - (NS - 1) * 624
    nst = slab // 40

    def dump(j, _):
        r = pl.ds(s * 624 + j * 40, 40)
        pltpu.sync_copy(acc_s.at[r], stage_v)
        pltpu.sync_copy(stage_v, out_hbm.at[c, r])
        return 0

    lax.fori_loop(0, nst, dump, 0)


def _scatter_sums_body(m2_hbm, dst_hbm, zn_hbm,
                       sums_hbm,
                       m20_v, m21_v, di0_v, di1_v, stage_v, acc_s,
                       sm0, sm1, si0, si1, sc0, sc1):
    e = dst_hbm.shape[0]
    n = zn_hbm.shape[0]
    epw = e // NW
    c = lax.axis_index("c")
    s = lax.axis_index("s")
    wid = s * NC + c
    _acc_init(zn_hbm, acc_s, stage_v, s, n)
    plsc.subcore_barrier()

    base0 = wid * epw
    nch = epw // CHUNK

    def pair(kk, _):
        base = base0 + kk * 2 * CHUNK
        ci0 = pltpu.async_copy(dst_hbm.at[pl.ds(base, CHUNK)], di0_v, si0)
        ci1 = pltpu.async_copy(
            dst_hbm.at[pl.ds(base + CHUNK, CHUNK)], di1_v, si1)
        cm0 = pltpu.async_copy(m2_hbm.at[pl.ds(base, CHUNK)], m20_v, sm0)
        cm1 = pltpu.async_copy(
            m2_hbm.at[pl.ds(base + CHUNK, CHUNK)], m21_v, sm1)
        ci0.wait()
        cm0.wait()
        a0 = pltpu.async_copy(m20_v, acc_s.at[di0_v], sc0, add=True)
        ci1.wait()
        cm1.wait()
        a1 = pltpu.async_copy(m21_v, acc_s.at[di1_v], sc1, add=True)
        a0.wait()
        a1.wait()
        return 0

    lax.fori_loop(0, nch // 2, pair, 0)

    base = base0 + (nch - 1) * CHUNK
    pltpu.sync_copy(dst_hbm.at[pl.ds(base, CHUNK)], di0_v)
    pltpu.sync_copy(m2_hbm.at[pl.ds(base, CHUNK)], m20_v)
    pltpu.sync_copy(m20_v, acc_s.at[di0_v], add=True)

    plsc.subcore_barrier()
    _acc_dump(acc_s, sums_hbm, stage_v, c, s, n)


def _scatter_counts_body(dst_hbm, zn_hbm, ones_hbm,
                         cnt_hbm,
                         ones_v, di0_v, di1_v, stage_v, acc_s,
                         si0, si1, sc0, sc1):
    e = dst_hbm.shape[0]
    n = zn_hbm.shape[0]
    epw = e // NW
    c = lax.axis_index("c")
    s = lax.axis_index("s")
    wid = s * NC + c
    _acc_init(zn_hbm, acc_s, stage_v, s, n)
    pltpu.sync_copy(ones_hbm, ones_v)
    plsc.subcore_barrier()

    base0 = wid * epw
    nch = epw // CHUNK

    def pair(kk, _):
        base = base0 + kk * 2 * CHUNK
        ci0 = pltpu.async_copy(dst_hbm.at[pl.ds(base, CHUNK)], di0_v, si0)
        ci1 = pltpu.async_copy(
            dst_hbm.at[pl.ds(base + CHUNK, CHUNK)], di1_v, si1)
        ci0.wait()
        a0 = pltpu.async_copy(ones_v, acc_s.at[di0_v], sc0, add=True)
        ci1.wait()
        a1 = pltpu.async_copy(ones_v, acc_s.at[di1_v], sc1, add=True)
        a0.wait()
        a1.wait()
        return 0

    lax.fori_loop(0, nch // 2, pair, 0)

    base = base0 + (nch - 1) * CHUNK
    pltpu.sync_copy(dst_hbm.at[pl.ds(base, CHUNK)], di0_v)
    pltpu.sync_copy(ones_v, acc_s.at[di0_v], add=True)

    plsc.subcore_barrier()
    _acc_dump(acc_s, cnt_hbm, stage_v, c, s, n)


def _scatter_sums(m2, dst, n):
    e, d = m2.shape
    zn = jnp.zeros((n, d), jnp.float32)
    return pl.kernel(
        _scatter_sums_body,
        out_type=jax.ShapeDtypeStruct((NC, n, d), jnp.float32),
        mesh=plsc.VectorSubcoreMesh(**_SC_MESH),
        scratch_types=[
            pltpu.VMEM((CHUNK, d), jnp.float32),
            pltpu.VMEM((CHUNK, d), jnp.float32),
            pltpu.VMEM((CHUNK,), jnp.int32),
            pltpu.VMEM((CHUNK,), jnp.int32),
            pltpu.VMEM((40, d), jnp.float32),
            pltpu.VMEM_SHARED((n, d), jnp.float32),
            pltpu.SemaphoreType.DMA,
            pltpu.SemaphoreType.DMA,
            pltpu.SemaphoreType.DMA,
            pltpu.SemaphoreType.DMA,
            pltpu.SemaphoreType.DMA,
            pltpu.SemaphoreType.DMA,
        ],
    )(m2, dst, zn)


def _scatter_counts(dst, n, d):
    zn = jnp.zeros((n, d), jnp.float32)
    ones = jnp.ones((CHUNK, d), jnp.float32)
    return pl.kernel(
        _scatter_counts_body,
        out_type=jax.ShapeDtypeStruct((NC, n, d), jnp.float32),
        mesh=plsc.VectorSubcoreMesh(**_SC_MESH),
        scratch_types=[
            pltpu.VMEM((CHUNK, d), jnp.float32),
            pltpu.VMEM((CHUNK,), jnp.int32),
            pltpu.VMEM((CHUNK,), jnp.int32),
            pltpu.VMEM((40, d), jnp.float32),
            pltpu.VMEM_SHARED((n, d), jnp.float32),
            pltpu.SemaphoreType.DMA,
            pltpu.SemaphoreType.DMA,
            pltpu.SemaphoreType.DMA,
            pltpu.SemaphoreType.DMA,
        ],
    )(dst, zn, ones)


def _node_upd_body(h_ref, hn_ref, sums_ref, cnt_ref, w1h_ref, w1m_ref,
                   bn1_ref, wn2_ref, bn2_ref, out_ref):
    cnt = cnt_ref[0, :, 0:1] + cnt_ref[1, :, 0:1]
    m = (sums_ref[0] + sums_ref[1]) / jnp.maximum(cnt, 1.0)
    t = (jnp.dot(hn_ref[...], w1h_ref[...], preferred_element_type=jnp.float32)
         + jnp.dot(m, w1m_ref[...], preferred_element_type=jnp.float32)
         + bn1_ref[...])
    t = t * jax.nn.sigmoid(t)
    y = jnp.dot(t, wn2_ref[...], preferred_element_type=jnp.float32) \
        + bn2_ref[...]
    out_ref[...] = h_ref[...] + y * jax.nn.sigmoid(y)


def _node_update(h, hn, sums, cnt, w1h, w1m, bn1, wn2, bn2, tn):
    n, d = h.shape
    grid = n // tn
    blk = lambda i: (i, 0)
    blk3 = lambda i: (0, i, 0)
    full = lambda i: (0, 0)
    return pl.pallas_call(
        _node_upd_body,
        grid=(grid,),
        in_specs=[pl.BlockSpec((tn, d), blk), pl.BlockSpec((tn, d), blk),
                  pl.BlockSpec((NC, tn, d), blk3),
                  pl.BlockSpec((NC, tn, d), blk3),
                  pl.BlockSpec((d, d), full), pl.BlockSpec((d, d), full),
                  pl.BlockSpec((1, d), full), pl.BlockSpec((d, d), full),
                  pl.BlockSpec((1, d), full)],
        out_specs=pl.BlockSpec((tn, d), blk),
        out_shape=jax.ShapeDtypeStruct((n, d), jnp.float32),
    )(h, hn, sums, cnt, w1h, w1m, bn1.reshape(1, d), wn2, bn2.reshape(1, d))


def kernel(h, rbf_edge, gram_edge, edge_index, ln_g, ln_b,
           We1, be1, We2, be2, Wn1, bn1, Wn2, bn2):
    n, d = h.shape
    src = edge_index[0]
    dst = edge_index[1]
    g = gram_edge.shape[1]

    w_src = We1[:d]
    w_dst = We1[d:2 * d]
    w_g = We1[2 * d:2 * d + g]
    w_r = We1[2 * d + g:]

    hn, a_proj, b_proj = _node_pre(h, ln_g, ln_b, w_src, w_dst, be1, tn=1000)
    gsum = _gather(a_proj, b_proj, src, dst)
    cnt = _scatter_counts(dst, n, d)
    m2 = _edge_mlp(gsum, gram_edge, rbf_edge, w_g, w_r, We2, be2,
                   te=2000)
    sums = _scatter_sums(m2, dst, n)
    return _node_update(h, hn, sums, cnt, Wn1[:d], Wn1[d:], bn1, Wn2, bn2,
                        tn=1000)
